# Initial kernel scaffold; baseline (speedup 1.0000x reference)
import jax
import jax.numpy as jnp
from jax.experimental import pallas as pl


def kernel(x_k, x_q, edge_index, point_centers_k, point_centers_q, x_edge, Wq, Wk, Wv, We, Wo, point_weights):
    raise NotImplementedError("write your pallas kernel here")


# speedup vs baseline: 16.6600x; 16.6600x over previous
import jax
import jax.numpy as jnp
from jax import lax
from jax.experimental import pallas as pl
from jax.experimental.pallas import tpu as pltpu
from jax.experimental.pallas import tpu_sc as plsc

FIBER_DIM = 128
HEADS = 16
POINT_DIM = 8
N = 10000
E = 320000
DISTANCE_SCALING = 10.0
SCALAR_SCALE = (2 * POINT_DIM) ** (-0.5)
POINT_SCALE = (2 * POINT_DIM * (9.0 / 2.0)) ** (-0.5)

NC = 2
NS = 16
NW = NC * NS
EPW = E // NW
BE = 40
NB = EPW // BE
NP_ = 10240
NPS = NP_ // NS

_F32 = jnp.float32


def _prep_body(xq_ref, xk_ref, pcq_ref, pck_ref, wq_ref, wk_ref, wv_ref,
               c2_ref, s_ref, a_ref, b_ref, nk_ref, *v_refs):
    bn = xq_ref.shape[0]
    wq = wq_ref[...]
    wk = wk_ref[...]
    wv = wv_ref[...]
    c2 = c2_ref[...]
    sel = s_ref[...]

    def mm(x, w):
        return lax.dot_general(x.reshape(bn * 4, 128), w,
                               (((1,), (0,)), ((), ())),
                               preferred_element_type=_F32).reshape(bn, 4, 128)

    q = mm(xq_ref[...], wq)
    k = mm(xk_ref[...], wk)
    v = mm(xk_ref[...], wv)
    cq = pcq_ref[...] / DISTANCE_SCALING
    ck = pck_ref[...] / DISTANCE_SCALING

    sq = q[:, 0, :]
    pq = q[:, 1:, :] + cq[:, :, None]
    sk = k[:, 0, :]
    pk = k[:, 1:, :] + ck[:, :, None]
    sv = v[:, 0, :]
    pv = v[:, 1:, :] + ck[:, :, None]

    a_ref[:, 0:1, :] = (sq * SCALAR_SCALE)[:, None, :]
    a_ref[:, 1:4, :] = pq * c2[None, :, :]
    b_ref[:, 0:1, :] = sk[:, None, :]
    b_ref[:, 1:4, :] = pk

    uk = (pk * pk) * c2[None, :, :]
    nk_ref[...] = 0.5 * lax.dot_general(uk[:, 0, :] + uk[:, 1, :] + uk[:, 2, :],
                                        sel, (((1,), (0,)), ((), ())),
                                        preferred_element_type=_F32)
    vs = (sv, pv[:, 0, :], pv[:, 1, :], pv[:, 2, :])
    for i in range(4):
        v_refs[2 * i][...] = vs[i][:, :64]
        v_refs[2 * i + 1][...] = vs[i][:, 64:]


def _prep(xq, xk, pcq, pck, wq, wk, wv, c2, sel):
    bn = 1000
    grid = N // bn
    full = lambda shp: pl.BlockSpec(shp, lambda i: tuple(0 for _ in shp))
    row3 = pl.BlockSpec((bn, 4, 128), lambda i: (i, 0, 0))
    return pl.pallas_call(
        _prep_body,
        grid=(grid,),
        in_specs=[row3, row3,
                  pl.BlockSpec((bn, 3), lambda i: (i, 0)),
                  pl.BlockSpec((bn, 3), lambda i: (i, 0)),
                  full((128, 128)), full((128, 128)), full((128, 128)),
                  full((1, 128)), full((128, 16))],
        out_specs=[row3, row3,
                   pl.BlockSpec((bn, 16), lambda i: (i, 0))] +
                  [pl.BlockSpec((bn, 64), lambda i: (i, 0))] * 8,
        out_shape=[jax.ShapeDtypeStruct((N, 4, 128), _F32),
                   jax.ShapeDtypeStruct((N, 4, 128), _F32),
                   jax.ShapeDtypeStruct((N, 16), _F32)] +
                  [jax.ShapeDtypeStruct((N, 64), _F32)] * 8,
    )(xq, xk, pcq, pck, wq, wk, wv, c2, sel)


def _ebias_body(xe_ref, we_ref, out_ref):
    out_ref[...] = lax.dot_general(xe_ref[...], we_ref[...],
                                   (((1,), (0,)), ((), ())),
                                   preferred_element_type=_F32)


def _ebias(x_edge, we):
    be = 8000
    return pl.pallas_call(
        _ebias_body,
        grid=(E // be,),
        in_specs=[pl.BlockSpec((be, 16), lambda i: (i, 0)),
                  pl.BlockSpec((16, 16), lambda i: (0, 0))],
        out_specs=pl.BlockSpec((be, 16), lambda i: (i, 0)),
        out_shape=jax.ShapeDtypeStruct((E, 16), _F32),
    )(x_edge, we)


def _sc_body(src_h, dst_h, a_h, b_h, nk_h, eb_h,
             v0_h, v1_h, v2_h, v3_h, v4_h, v5_h, v6_h, v7_h, zd_h, za_h,
             den_out, acc_out, ex_out,
             idx_s, idx_d, arows, brows, nkr, ebr, exb, vrows,
             den_sp, acc_sp, sem0, sem1, sem2):
    c = lax.axis_index("c")
    s = lax.axis_index("s")
    wid = s * NC + c
    ebase = wid * EPW
    rbase = s * NPS

    pltpu.sync_copy(zd_h, den_sp.at[pl.ds(rbase, NPS)])
    plsc.subcore_barrier()

    def blk_a(i, carry):
        base = ebase + i * BE
        pltpu.sync_copy(src_h.at[pl.ds(base, BE)], idx_s)
        pltpu.sync_copy(dst_h.at[pl.ds(base, BE)], idx_d)
        cp_a = pltpu.async_copy(a_h.at[idx_d], arows, sem0)
        cp_b = pltpu.async_copy(b_h.at[idx_s], brows, sem1)
        cp_nk = pltpu.async_copy(nk_h.at[idx_s], nkr, sem2)
        pltpu.sync_copy(eb_h.at[pl.ds(base, BE)], ebr)
        cp_a.wait()
        cp_b.wait()
        cp_nk.wait()

        def edge(j, carry2):
            lg = ebr[j] - nkr[j]
            for t in range(32):
                lg = lg + arows[j, pl.ds(t * 16, 16)] * brows[j, pl.ds(t * 16, 16)]
            exb[j] = jnp.exp(lg)
            return carry2

        lax.fori_loop(0, BE, edge, 0, unroll=False)
        pltpu.sync_copy(exb, ex_out.at[pl.ds(base, BE)])
        pltpu.sync_copy(exb, den_sp.at[idx_d], add=True)
        return carry

    lax.fori_loop(0, NB, blk_a, 0, unroll=False)
    plsc.subcore_barrier()
    pltpu.sync_copy(den_sp.at[pl.ds(rbase, NPS)],
                    den_out.at[c, pl.ds(rbase, NPS)])

    for part, v_h in enumerate((v0_h, v1_h, v2_h, v3_h,
                                v4_h, v5_h, v6_h, v7_h)):
        plsc.subcore_barrier()
        pltpu.sync_copy(za_h, acc_sp.at[pl.ds(rbase, NPS)])
        plsc.subcore_barrier()

        def blk_b(i, carry):
            base = ebase + i * BE
            pltpu.sync_copy(src_h.at[pl.ds(base, BE)], idx_s)
            pltpu.sync_copy(dst_h.at[pl.ds(base, BE)], idx_d)
            cp_v = pltpu.async_copy(v_h.at[idx_s], vrows, sem0)
            pltpu.sync_copy(ex_out.at[pl.ds(base, BE)], exb)
            cp_v.wait()

            def edge(j, carry2):
                al = exb[j]
                for t in range(4):
                    vrows[j, pl.ds(t * 16, 16)] = vrows[j, pl.ds(t * 16, 16)] * al
                return carry2

            lax.fori_loop(0, BE, edge, 0, unroll=False)
            pltpu.sync_copy(vrows, acc_sp.at[idx_d], add=True)
            return carry

        lax.fori_loop(0, NB, blk_b, 0, unroll=False)
        plsc.subcore_barrier()
        pltpu.sync_copy(acc_sp.at[pl.ds(rbase, NPS)],
                        acc_out.at[part, c, pl.ds(rbase, NPS)])
    plsc.subcore_barrier()


def _sc_edge(src, dst, a, b, nk, eb, vs, zd, za):
    mesh = plsc.VectorSubcoreMesh(core_axis_name="c", subcore_axis_name="s")
    f = pl.kernel(
        _sc_body,
        out_type=[jax.ShapeDtypeStruct((NC, NP_, 16), _F32),
                  jax.ShapeDtypeStruct((8, NC, NP_, 64), _F32),
                  jax.ShapeDtypeStruct((E, 16), _F32)],
        mesh=mesh,
        compiler_params=pltpu.CompilerParams(use_tc_tiling_on_sc=False),
        scratch_types=[
            pltpu.VMEM((BE,), jnp.int32),
            pltpu.VMEM((BE,), jnp.int32),
            pltpu.VMEM((BE, 512), _F32),
            pltpu.VMEM((BE, 512), _F32),
            pltpu.VMEM((BE, 16), _F32),
            pltpu.VMEM((BE, 16), _F32),
            pltpu.VMEM((BE, 16), _F32),
            pltpu.VMEM((BE, 64), _F32),
            pltpu.VMEM_SHARED((NP_, 16), _F32),
            pltpu.VMEM_SHARED((NP_, 64), _F32),
            pltpu.SemaphoreType.DMA,
            pltpu.SemaphoreType.DMA,
            pltpu.SemaphoreType.DMA,
        ],
    )
    return f(src, dst, a, b, nk, eb, *vs, zd, za)


def _final_body(den_ref, acc_ref, pcq_ref, st_ref, wo_ref, out_ref):
    bn = pcq_ref.shape[0]
    d = den_ref[0] + den_ref[1] + 1e-16
    rinv = lax.dot_general(1.0 / d, st_ref[...],
                           (((1,), (0,)), ((), ())),
                           preferred_element_type=_F32)
    cq = pcq_ref[...] / DISTANCE_SCALING
    wo = wo_ref[...]
    parts = []
    for i in range(4):
        h0 = (acc_ref[2 * i, 0] + acc_ref[2 * i, 1]) * rinv[:, :64]
        h1 = (acc_ref[2 * i + 1, 0] + acc_ref[2 * i + 1, 1]) * rinv[:, 64:]
        if i > 0:
            h0 = h0 - cq[:, i - 1][:, None]
            h1 = h1 - cq[:, i - 1][:, None]
        o = (lax.dot_general(h0, wo[:64], (((1,), (0,)), ((), ())),
                             preferred_element_type=_F32) +
             lax.dot_general(h1, wo[64:], (((1,), (0,)), ((), ())),
                             preferred_element_type=_F32))
        parts.append(o[:, None, :])
    out_ref[...] = jnp.concatenate(parts, axis=1)


def _finalize(den2, acc4, pcq, st, wo):
    bn = 1000
    return pl.pallas_call(
        _final_body,
        grid=(N // bn,),
        in_specs=[pl.BlockSpec((2, bn, 16), lambda i: (0, i, 0)),
                  pl.BlockSpec((8, 2, bn, 64), lambda i: (0, 0, i, 0)),
                  pl.BlockSpec((bn, 3), lambda i: (i, 0)),
                  pl.BlockSpec((16, 128), lambda i: (0, 0)),
                  pl.BlockSpec((128, 128), lambda i: (0, 0))],
        out_specs=pl.BlockSpec((bn, 4, 128), lambda i: (i, 0, 0)),
        out_shape=jax.ShapeDtypeStruct((N, 4, 128), _F32),
    )(den2, acc4, pcq, st, wo)


def kernel(x_k, x_q, edge_index, point_centers_k, point_centers_q, x_edge,
           Wq, Wk, Wv, We, Wo, point_weights):
    perm = jnp.array([h * POINT_DIM + p
                      for p in range(POINT_DIM) for h in range(HEADS)],
                     dtype=jnp.int32)
    wq_p = Wq[:, perm]
    wk_p = Wk[:, perm]
    wv_p = Wv[:, perm]
    wo_p = Wo[perm, :]
    pw = jax.nn.softplus(point_weights)
    c2 = (jnp.tile(pw, POINT_DIM) * POINT_SCALE).reshape(1, 128)
    sel = jnp.tile(jnp.eye(16, dtype=_F32), (POINT_DIM, 1))

    a, b, nk, *vs = _prep(
        x_q, x_k, point_centers_q, point_centers_k, wq_p, wk_p, wv_p, c2, sel)
    eb = _ebias(x_edge, We)

    src = edge_index[0]
    dst = edge_index[1]
    zd = jnp.zeros((NPS, 16), _F32)
    za = jnp.zeros((NPS, 64), _F32)
    den2, acc8, _ex = _sc_edge(
        src, dst, a.reshape(N, 512), b.reshape(N, 512), nk, eb, vs, zd, za)

    return _finalize(den2[:, :N], acc8[:, :, :N], point_centers_q, sel.T, wo_p)


# speedup vs baseline: 24.5015x; 1.4707x over previous
import jax
import jax.numpy as jnp
from jax import lax
from jax.experimental import pallas as pl
from jax.experimental.pallas import tpu as pltpu
from jax.experimental.pallas import tpu_sc as plsc

FIBER_DIM = 128
HEADS = 16
POINT_DIM = 8
N = 10000
E = 320000
DISTANCE_SCALING = 10.0
SCALAR_SCALE = (2 * POINT_DIM) ** (-0.5)
POINT_SCALE = (2 * POINT_DIM * (9.0 / 2.0)) ** (-0.5)

NC = 2
NS = 16
NW = NC * NS
EPW = E // NW
BE = 40
NB = EPW // BE
NP_ = 10240
NPS = NP_ // NS

_F32 = jnp.float32


def _prep_body(xq_ref, xk_ref, pcq_ref, pck_ref, wq_ref, wk_ref, wv_ref,
               c2_ref, s_ref, alo_ref, ahi_ref, blo_ref, bhi_ref, nk_ref,
               *v_refs):
    bn = xq_ref.shape[0]
    wq = wq_ref[...]
    wk = wk_ref[...]
    wv = wv_ref[...]
    c2 = c2_ref[...]
    sel = s_ref[...]

    def mm(x, w):
        return lax.dot_general(x.reshape(bn * 4, 128), w,
                               (((1,), (0,)), ((), ())),
                               preferred_element_type=_F32).reshape(bn, 4, 128)

    q = mm(xq_ref[...], wq)
    k = mm(xk_ref[...], wk)
    v = mm(xk_ref[...], wv)
    cq = pcq_ref[...] / DISTANCE_SCALING
    ck = pck_ref[...] / DISTANCE_SCALING

    sq = q[:, 0, :]
    pq = q[:, 1:, :] + cq[:, :, None]
    sk = k[:, 0, :]
    pk = k[:, 1:, :] + ck[:, :, None]
    sv = v[:, 0, :]
    pv = v[:, 1:, :] + ck[:, :, None]

    ap = pq * c2[None, :, :]
    alo_ref[:, 0:1, :] = (sq * SCALAR_SCALE)[:, None, :]
    alo_ref[:, 1:2, :] = ap[:, 0:1, :]
    ahi_ref[...] = ap[:, 1:3, :]
    blo_ref[:, 0:1, :] = sk[:, None, :]
    blo_ref[:, 1:2, :] = pk[:, 0:1, :]
    bhi_ref[...] = pk[:, 1:3, :]

    uk = (pk * pk) * c2[None, :, :]
    nk_ref[...] = 0.5 * lax.dot_general(uk[:, 0, :] + uk[:, 1, :] + uk[:, 2, :],
                                        sel, (((1,), (0,)), ((), ())),
                                        preferred_element_type=_F32)
    vs = (sv, pv[:, 0, :], pv[:, 1, :], pv[:, 2, :])
    for i in range(4):
        v_refs[i][...] = vs[i]


def _prep(xq, xk, pcq, pck, wq, wk, wv, c2, sel):
    bn = 1000
    grid = N // bn
    full = lambda shp: pl.BlockSpec(shp, lambda i: tuple(0 for _ in shp))
    row3 = pl.BlockSpec((bn, 4, 128), lambda i: (i, 0, 0))
    return pl.pallas_call(
        _prep_body,
        grid=(grid,),
        in_specs=[row3, row3,
                  pl.BlockSpec((bn, 3), lambda i: (i, 0)),
                  pl.BlockSpec((bn, 3), lambda i: (i, 0)),
                  full((128, 128)), full((128, 128)), full((128, 128)),
                  full((1, 128)), full((128, 16))],
        out_specs=[pl.BlockSpec((bn, 2, 128), lambda i: (i, 0, 0))] * 4 +
                  [pl.BlockSpec((bn, 16), lambda i: (i, 0))] +
                  [pl.BlockSpec((bn, 128), lambda i: (i, 0))] * 4,
        out_shape=[jax.ShapeDtypeStruct((N, 2, 128), _F32)] * 4 +
                  [jax.ShapeDtypeStruct((N, 16), _F32)] +
                  [jax.ShapeDtypeStruct((N, 128), _F32)] * 4,
    )(xq, xk, pcq, pck, wq, wk, wv, c2, sel)


def _ebias_body(xe_ref, we_ref, out_ref):
    out_ref[...] = lax.dot_general(xe_ref[...], we_ref[...],
                                   (((1,), (0,)), ((), ())),
                                   preferred_element_type=_F32)


def _ebias(x_edge, we):
    be = 8000
    return pl.pallas_call(
        _ebias_body,
        grid=(E // be,),
        in_specs=[pl.BlockSpec((be, 16), lambda i: (i, 0)),
                  pl.BlockSpec((16, 16), lambda i: (0, 0))],
        out_specs=pl.BlockSpec((be, 16), lambda i: (i, 0)),
        out_shape=jax.ShapeDtypeStruct((E, 16), _F32),
    )(x_edge, we)


def _sc_body(src_h, dst_h, alo_h, ahi_h, blo_h, bhi_h, nk_h, eb_h,
             v0_h, v1_h, v2_h, v3_h, zd_h, za_h,
             den_out, acc_out, ex_out,
             ixs0, ixd0, ixs1, ixd1, exq0, exq1, nkr, ebr, exb,
             den_sp, acc_sp, sem0, sem1, sem2):
    c = lax.axis_index("c")
    s = lax.axis_index("s")
    wid = s * NC + c
    ebase = wid * EPW
    rbase = s * NPS

    def main(arows, brows, vr0, vr1, plog):
        pltpu.sync_copy(zd_h, den_sp.at[pl.ds(rbase, NPS)])
        plsc.subcore_barrier()

        def blk_a(i, carry):
            base = ebase + i * BE
            pltpu.sync_copy(src_h.at[pl.ds(base, BE)], ixs0)
            pltpu.sync_copy(dst_h.at[pl.ds(base, BE)], ixd0)
            cp_a = pltpu.async_copy(alo_h.at[ixd0], arows, sem0)
            cp_b = pltpu.async_copy(blo_h.at[ixs0], brows, sem1)
            cp_nk = pltpu.async_copy(nk_h.at[ixs0], nkr, sem2)
            pltpu.sync_copy(eb_h.at[pl.ds(base, BE)], ebr)
            cp_a.wait()
            cp_b.wait()

            def edge_lo(j, carry2):
                ps = [arows[j, pl.ds(0, 16)] * brows[j, pl.ds(0, 16)],
                      arows[j, pl.ds(16, 16)] * brows[j, pl.ds(16, 16)],
                      arows[j, pl.ds(32, 16)] * brows[j, pl.ds(32, 16)],
                      arows[j, pl.ds(48, 16)] * brows[j, pl.ds(48, 16)]]
                for t in range(4, 16):
                    o = t * 16
                    ps[t % 4] = ps[t % 4] + arows[j, pl.ds(o, 16)] * brows[j, pl.ds(o, 16)]
                plog[j] = (ps[0] + ps[1]) + (ps[2] + ps[3])
                return carry2

            lax.fori_loop(0, BE, edge_lo, 0, unroll=False)
            cp_a2 = pltpu.async_copy(ahi_h.at[ixd0], arows, sem0)
            cp_b2 = pltpu.async_copy(bhi_h.at[ixs0], brows, sem1)
            cp_nk.wait()
            cp_a2.wait()
            cp_b2.wait()

            def edge_hi(j, carry2):
                ps = [plog[j] + ebr[j] - nkr[j],
                      arows[j, pl.ds(0, 16)] * brows[j, pl.ds(0, 16)],
                      arows[j, pl.ds(16, 16)] * brows[j, pl.ds(16, 16)],
                      arows[j, pl.ds(32, 16)] * brows[j, pl.ds(32, 16)]]
                for t in range(3, 16):
                    o = t * 16
                    ps[t % 4] = ps[t % 4] + arows[j, pl.ds(o, 16)] * brows[j, pl.ds(o, 16)]
                exb[j] = jnp.exp((ps[0] + ps[1]) + (ps[2] + ps[3]))
                return carry2

            lax.fori_loop(0, BE, edge_hi, 0, unroll=False)
            pltpu.sync_copy(exb, ex_out.at[pl.ds(base, BE)])
            pltpu.sync_copy(exb, den_sp.at[ixd0], add=True)
            return carry

        lax.fori_loop(0, NB, blk_a, 0, unroll=False)
        plsc.subcore_barrier()
        pltpu.sync_copy(den_sp.at[pl.ds(rbase, NPS)],
                        den_out.at[c, pl.ds(rbase, NPS)])

        bufs = ((vr0, ixs0, ixd0, exq0, sem0), (vr1, ixs1, ixd1, exq1, sem1))
        for part, v_h in enumerate((v0_h, v1_h, v2_h, v3_h)):
            plsc.subcore_barrier()
            pltpu.sync_copy(za_h, acc_sp.at[pl.ds(rbase, NPS)])
            plsc.subcore_barrier()

            pltpu.sync_copy(src_h.at[pl.ds(ebase, BE)], ixs0)
            pltpu.sync_copy(dst_h.at[pl.ds(ebase, BE)], ixd0)
            pltpu.sync_copy(ex_out.at[pl.ds(ebase, BE)], exq0)
            pltpu.async_copy(v_h.at[ixs0], vr0, sem0)

            def blk2(g, carry):
                for b in (0, 1):
                    vr, ixs, ixd, exq, sem = bufs[b]
                    vrn, ixsn, ixdn, exqn, semn = bufs[1 - b]
                    i = 2 * g + b
                    nxt = ebase + jnp.minimum(i + 1, NB - 1) * BE
                    pltpu.sync_copy(src_h.at[pl.ds(nxt, BE)], ixsn)
                    pltpu.sync_copy(dst_h.at[pl.ds(nxt, BE)], ixdn)
                    pltpu.sync_copy(ex_out.at[pl.ds(nxt, BE)], exqn)
                    pltpu.async_copy(v_h.at[ixsn], vrn, semn)

                    pltpu.make_async_copy(v_h.at[ixs], vr, sem).wait()

                    def edge(j, carry2):
                        al = exq[j]
                        for t in range(8):
                            o = t * 16
                            vr[j, pl.ds(o, 16)] = vr[j, pl.ds(o, 16)] * al
                        return carry2

                    lax.fori_loop(0, BE, edge, 0, unroll=False)
                    pltpu.sync_copy(vr, acc_sp.at[ixd], add=True)
                return carry

            lax.fori_loop(0, NB // 2, blk2, 0, unroll=False)
            pltpu.make_async_copy(v_h.at[ixs0], vr0, sem0).wait()

            plsc.subcore_barrier()
            pltpu.sync_copy(acc_sp.at[pl.ds(rbase, NPS)],
                            acc_out.at[part, c, pl.ds(rbase, NPS)])
        plsc.subcore_barrier()

    pl.run_scoped(main,
                  pltpu.VMEM((BE, 256), _F32), pltpu.VMEM((BE, 256), _F32),
                  pltpu.VMEM((BE, 128), _F32), pltpu.VMEM((BE, 128), _F32),
                  pltpu.VMEM((BE, 16), _F32))


def _sc_edge(src, dst, alo, ahi, blo, bhi, nk, eb, vs, zd, za):
    mesh = plsc.VectorSubcoreMesh(core_axis_name="c", subcore_axis_name="s")
    f = pl.kernel(
        _sc_body,
        out_type=[jax.ShapeDtypeStruct((NC, NP_, 16), _F32),
                  jax.ShapeDtypeStruct((4, NC, NP_, 128), _F32),
                  jax.ShapeDtypeStruct((E, 16), _F32)],
        mesh=mesh,
        compiler_params=pltpu.CompilerParams(use_tc_tiling_on_sc=False),
        scratch_types=[
            pltpu.VMEM((BE,), jnp.int32), pltpu.VMEM((BE,), jnp.int32),
            pltpu.VMEM((BE,), jnp.int32), pltpu.VMEM((BE,), jnp.int32),
            pltpu.VMEM((BE, 16), _F32), pltpu.VMEM((BE, 16), _F32),
            pltpu.VMEM((BE, 16), _F32), pltpu.VMEM((BE, 16), _F32),
            pltpu.VMEM((BE, 16), _F32),
            pltpu.VMEM_SHARED((NP_, 16), _F32),
            pltpu.VMEM_SHARED((NP_, 128), _F32),
            pltpu.SemaphoreType.DMA,
            pltpu.SemaphoreType.DMA,
            pltpu.SemaphoreType.DMA,
        ],
    )
    return f(src, dst, alo, ahi, blo, bhi, nk, eb, *vs, zd, za)


def _final_body(den_ref, acc_ref, pcq_ref, st_ref, wo_ref, out_ref):
    bn = pcq_ref.shape[0]
    d = den_ref[0] + den_ref[1] + 1e-16
    rinv = lax.dot_general(1.0 / d, st_ref[...],
                           (((1,), (0,)), ((), ())),
                           preferred_element_type=_F32)
    cq = pcq_ref[...] / DISTANCE_SCALING
    wo = wo_ref[...]
    parts = []
    for i in range(4):
        r = (acc_ref[i, 0] + acc_ref[i, 1]) * rinv
        if i > 0:
            r = r - cq[:, i - 1][:, None]
        o = lax.dot_general(r, wo, (((1,), (0,)), ((), ())),
                            preferred_element_type=_F32)
        parts.append(o[:, None, :])
    out_ref[...] = jnp.concatenate(parts, axis=1)


def _finalize(den2, acc4, pcq, st, wo):
    bn = 1000
    return pl.pallas_call(
        _final_body,
        grid=(N // bn,),
        in_specs=[pl.BlockSpec((2, bn, 16), lambda i: (0, i, 0)),
                  pl.BlockSpec((4, 2, bn, 128), lambda i: (0, 0, i, 0)),
                  pl.BlockSpec((bn, 3), lambda i: (i, 0)),
                  pl.BlockSpec((16, 128), lambda i: (0, 0)),
                  pl.BlockSpec((128, 128), lambda i: (0, 0))],
        out_specs=pl.BlockSpec((bn, 4, 128), lambda i: (i, 0, 0)),
        out_shape=jax.ShapeDtypeStruct((N, 4, 128), _F32),
    )(den2, acc4, pcq, st, wo)


def kernel(x_k, x_q, edge_index, point_centers_k, point_centers_q, x_edge,
           Wq, Wk, Wv, We, Wo, point_weights):
    perm = jnp.array([h * POINT_DIM + p
                      for p in range(POINT_DIM) for h in range(HEADS)],
                     dtype=jnp.int32)
    wq_p = Wq[:, perm]
    wk_p = Wk[:, perm]
    wv_p = Wv[:, perm]
    wo_p = Wo[perm, :]
    pw = jax.nn.softplus(point_weights)
    c2 = (jnp.tile(pw, POINT_DIM) * POINT_SCALE).reshape(1, 128)
    sel = jnp.tile(jnp.eye(16, dtype=_F32), (POINT_DIM, 1))

    alo, ahi, blo, bhi, nk, *vs = _prep(
        x_q, x_k, point_centers_q, point_centers_k, wq_p, wk_p, wv_p, c2, sel)
    eb = _ebias(x_edge, We)

    src = edge_index[0]
    dst = edge_index[1]
    zd = jnp.zeros((NPS, 16), _F32)
    za = jnp.zeros((NPS, 128), _F32)
    den2, acc4, _ex = _sc_edge(
        src, dst, alo.reshape(N, 256), ahi.reshape(N, 256),
        blo.reshape(N, 256), bhi.reshape(N, 256), nk, eb, vs, zd, za)

    return _finalize(den2[:, :N], acc4[:, :, :N], point_centers_q, sel.T, wo_p)


# speedup vs baseline: 26.4110x; 1.0779x over previous
import jax
import jax.numpy as jnp
from jax import lax
from jax.experimental import pallas as pl
from jax.experimental.pallas import tpu as pltpu
from jax.experimental.pallas import tpu_sc as plsc

FIBER_DIM = 128
HEADS = 16
POINT_DIM = 8
N = 10000
E = 320000
DISTANCE_SCALING = 10.0
SCALAR_SCALE = (2 * POINT_DIM) ** (-0.5)
POINT_SCALE = (2 * POINT_DIM * (9.0 / 2.0)) ** (-0.5)

NC = 2
NS = 16
NW = NC * NS
EPW = E // NW
BE = 40
NB = EPW // BE
NP_ = 10240
NPS = NP_ // NS

_F32 = jnp.float32


def _prep_body(xq_ref, xk_ref, pcq_ref, pck_ref, wq_ref, wk_ref, wv_ref,
               c2_ref, s_ref, alo_ref, ahi_ref, blo_ref, bhi_ref, nk_ref,
               *v_refs):
    bn = xq_ref.shape[0]
    wq = wq_ref[...]
    wk = wk_ref[...]
    wv = wv_ref[...]
    c2 = c2_ref[...]
    sel = s_ref[...]

    def mm(x, w):
        return lax.dot_general(x.reshape(bn * 4, 128), w,
                               (((1,), (0,)), ((), ())),
                               preferred_element_type=_F32).reshape(bn, 4, 128)

    q = mm(xq_ref[...], wq)
    k = mm(xk_ref[...], wk)
    v = mm(xk_ref[...], wv)
    cq = pcq_ref[...] / DISTANCE_SCALING
    ck = pck_ref[...] / DISTANCE_SCALING

    sq = q[:, 0, :]
    pq = q[:, 1:, :] + cq[:, :, None]
    sk = k[:, 0, :]
    pk = k[:, 1:, :] + ck[:, :, None]
    sv = v[:, 0, :]
    pv = v[:, 1:, :] + ck[:, :, None]

    ap = pq * c2[None, :, :]
    alo_ref[:, 0:1, :] = (sq * SCALAR_SCALE)[:, None, :]
    alo_ref[:, 1:2, :] = ap[:, 0:1, :]
    ahi_ref[...] = ap[:, 1:3, :]
    blo_ref[:, 0:1, :] = sk[:, None, :]
    blo_ref[:, 1:2, :] = pk[:, 0:1, :]
    bhi_ref[...] = pk[:, 1:3, :]

    uk = (pk * pk) * c2[None, :, :]
    nk_ref[...] = 0.5 * lax.dot_general(uk[:, 0, :] + uk[:, 1, :] + uk[:, 2, :],
                                        sel, (((1,), (0,)), ((), ())),
                                        preferred_element_type=_F32)
    vs = (sv, pv[:, 0, :], pv[:, 1, :], pv[:, 2, :])
    for i in range(4):
        v_refs[i][...] = vs[i]


def _prep(xq, xk, pcq, pck, wq, wk, wv, c2, sel):
    bn = 1000
    grid = N // bn
    full = lambda shp: pl.BlockSpec(shp, lambda i: tuple(0 for _ in shp))
    row3 = pl.BlockSpec((bn, 4, 128), lambda i: (i, 0, 0))
    return pl.pallas_call(
        _prep_body,
        grid=(grid,),
        in_specs=[row3, row3,
                  pl.BlockSpec((bn, 3), lambda i: (i, 0)),
                  pl.BlockSpec((bn, 3), lambda i: (i, 0)),
                  full((128, 128)), full((128, 128)), full((128, 128)),
                  full((1, 128)), full((128, 16))],
        out_specs=[pl.BlockSpec((bn, 2, 128), lambda i: (i, 0, 0))] * 4 +
                  [pl.BlockSpec((bn, 16), lambda i: (i, 0))] +
                  [pl.BlockSpec((bn, 128), lambda i: (i, 0))] * 4,
        out_shape=[jax.ShapeDtypeStruct((N, 2, 128), _F32)] * 4 +
                  [jax.ShapeDtypeStruct((N, 16), _F32)] +
                  [jax.ShapeDtypeStruct((N, 128), _F32)] * 4,
    )(xq, xk, pcq, pck, wq, wk, wv, c2, sel)


def _ebias_body(xe_ref, we_ref, out_ref):
    out_ref[...] = lax.dot_general(xe_ref[...], we_ref[...],
                                   (((1,), (0,)), ((), ())),
                                   preferred_element_type=_F32)


def _ebias(x_edge, we):
    be = 8000
    return pl.pallas_call(
        _ebias_body,
        grid=(E // be,),
        in_specs=[pl.BlockSpec((be, 16), lambda i: (i, 0)),
                  pl.BlockSpec((16, 16), lambda i: (0, 0))],
        out_specs=pl.BlockSpec((be, 16), lambda i: (i, 0)),
        out_shape=jax.ShapeDtypeStruct((E, 16), _F32),
    )(x_edge, we)


def _sc_body(src_h, dst_h, alo_h, ahi_h, blo_h, bhi_h, nk_h, eb_h,
             v0_h, v1_h, v2_h, v3_h, zd_h, za_h,
             den_out, acc_out, ex_out,
             ixs0, ixd0, ixs1, ixd1, exq0, exq1, nkr, ebr, exb,
             den_sp, acc_sp, sem0, sem1, sem2):
    c = lax.axis_index("c")
    s = lax.axis_index("s")
    wid = s * NC + c
    ebase = wid * EPW
    rbase = s * NPS

    def main(arows, brows, vr0, vr1, plog):
        pltpu.sync_copy(zd_h, den_sp.at[pl.ds(rbase, NPS)])
        plsc.subcore_barrier()

        def blk_a(i, carry):
            base = ebase + i * BE
            pltpu.sync_copy(src_h.at[pl.ds(base, BE)], ixs0)
            pltpu.sync_copy(dst_h.at[pl.ds(base, BE)], ixd0)
            cp_a = pltpu.async_copy(alo_h.at[ixd0], arows, sem0)
            cp_b = pltpu.async_copy(blo_h.at[ixs0], brows, sem1)
            cp_nk = pltpu.async_copy(nk_h.at[ixs0], nkr, sem2)
            pltpu.sync_copy(eb_h.at[pl.ds(base, BE)], ebr)
            cp_a.wait()
            cp_b.wait()

            @plsc.parallel_loop(0, BE)
            def edge_lo(j):
                ps = [arows[j, pl.ds(0, 16)] * brows[j, pl.ds(0, 16)],
                      arows[j, pl.ds(16, 16)] * brows[j, pl.ds(16, 16)],
                      arows[j, pl.ds(32, 16)] * brows[j, pl.ds(32, 16)],
                      arows[j, pl.ds(48, 16)] * brows[j, pl.ds(48, 16)]]
                for t in range(4, 16):
                    o = t * 16
                    ps[t % 4] = ps[t % 4] + arows[j, pl.ds(o, 16)] * brows[j, pl.ds(o, 16)]
                plog[j] = (ps[0] + ps[1]) + (ps[2] + ps[3])

            cp_a2 = pltpu.async_copy(ahi_h.at[ixd0], arows, sem0)
            cp_b2 = pltpu.async_copy(bhi_h.at[ixs0], brows, sem1)
            cp_nk.wait()
            cp_a2.wait()
            cp_b2.wait()

            @plsc.parallel_loop(0, BE)
            def edge_hi(j):
                ps = [plog[j] + ebr[j] - nkr[j],
                      arows[j, pl.ds(0, 16)] * brows[j, pl.ds(0, 16)],
                      arows[j, pl.ds(16, 16)] * brows[j, pl.ds(16, 16)],
                      arows[j, pl.ds(32, 16)] * brows[j, pl.ds(32, 16)]]
                for t in range(3, 16):
                    o = t * 16
                    ps[t % 4] = ps[t % 4] + arows[j, pl.ds(o, 16)] * brows[j, pl.ds(o, 16)]
                exb[j] = jnp.exp((ps[0] + ps[1]) + (ps[2] + ps[3]))

            pltpu.sync_copy(exb, ex_out.at[pl.ds(base, BE)])
            pltpu.sync_copy(exb, den_sp.at[ixd0], add=True)
            return carry

        lax.fori_loop(0, NB, blk_a, 0, unroll=False)
        plsc.subcore_barrier()
        pltpu.sync_copy(den_sp.at[pl.ds(rbase, NPS)],
                        den_out.at[c, pl.ds(rbase, NPS)])

        bufs = ((vr0, ixs0, ixd0, exq0, sem0), (vr1, ixs1, ixd1, exq1, sem1))
        for part, v_h in enumerate((v0_h, v1_h, v2_h, v3_h)):
            plsc.subcore_barrier()
            pltpu.sync_copy(za_h, acc_sp.at[pl.ds(rbase, NPS)])
            plsc.subcore_barrier()

            pltpu.sync_copy(src_h.at[pl.ds(ebase, BE)], ixs0)
            pltpu.sync_copy(dst_h.at[pl.ds(ebase, BE)], ixd0)
            pltpu.sync_copy(ex_out.at[pl.ds(ebase, BE)], exq0)
            pltpu.async_copy(v_h.at[ixs0], vr0, sem0)

            def blk2(g, carry):
                for b in (0, 1):
                    vr, ixs, ixd, exq, sem = bufs[b]
                    vrn, ixsn, ixdn, exqn, semn = bufs[1 - b]
                    i = 2 * g + b
                    nxt = ebase + jnp.minimum(i + 1, NB - 1) * BE
                    pltpu.sync_copy(src_h.at[pl.ds(nxt, BE)], ixsn)
                    pltpu.sync_copy(dst_h.at[pl.ds(nxt, BE)], ixdn)
                    pltpu.sync_copy(ex_out.at[pl.ds(nxt, BE)], exqn)
                    pltpu.async_copy(v_h.at[ixsn], vrn, semn)

                    pltpu.make_async_copy(v_h.at[ixs], vr, sem).wait()

                    @plsc.parallel_loop(0, BE)
                    def edge(j):
                        al = exq[j]
                        for t in range(8):
                            o = t * 16
                            vr[j, pl.ds(o, 16)] = vr[j, pl.ds(o, 16)] * al

                    pltpu.sync_copy(vr, acc_sp.at[ixd], add=True)
                return carry

            lax.fori_loop(0, NB // 2, blk2, 0, unroll=False)
            pltpu.make_async_copy(v_h.at[ixs0], vr0, sem0).wait()

            plsc.subcore_barrier()
            pltpu.sync_copy(acc_sp.at[pl.ds(rbase, NPS)],
                            acc_out.at[part, c, pl.ds(rbase, NPS)])
        plsc.subcore_barrier()

    pl.run_scoped(main,
                  pltpu.VMEM((BE, 256), _F32), pltpu.VMEM((BE, 256), _F32),
                  pltpu.VMEM((BE, 128), _F32), pltpu.VMEM((BE, 128), _F32),
                  pltpu.VMEM((BE, 16), _F32))


def _sc_edge(src, dst, alo, ahi, blo, bhi, nk, eb, vs, zd, za):
    mesh = plsc.VectorSubcoreMesh(core_axis_name="c", subcore_axis_name="s")
    f = pl.kernel(
        _sc_body,
        out_type=[jax.ShapeDtypeStruct((NC, NP_, 16), _F32),
                  jax.ShapeDtypeStruct((4, NC, NP_, 128), _F32),
                  jax.ShapeDtypeStruct((E, 16), _F32)],
        mesh=mesh,
        compiler_params=pltpu.CompilerParams(use_tc_tiling_on_sc=False),
        scratch_types=[
            pltpu.VMEM((BE,), jnp.int32), pltpu.VMEM((BE,), jnp.int32),
            pltpu.VMEM((BE,), jnp.int32), pltpu.VMEM((BE,), jnp.int32),
            pltpu.VMEM((BE, 16), _F32), pltpu.VMEM((BE, 16), _F32),
            pltpu.VMEM((BE, 16), _F32), pltpu.VMEM((BE, 16), _F32),
            pltpu.VMEM((BE, 16), _F32),
            pltpu.VMEM_SHARED((NP_, 16), _F32),
            pltpu.VMEM_SHARED((NP_, 128), _F32),
            pltpu.SemaphoreType.DMA,
            pltpu.SemaphoreType.DMA,
            pltpu.SemaphoreType.DMA,
        ],
    )
    return f(src, dst, alo, ahi, blo, bhi, nk, eb, *vs, zd, za)


def _final_body(den_ref, acc_ref, pcq_ref, st_ref, wo_ref, out_ref):
    bn = pcq_ref.shape[0]
    d = den_ref[0] + den_ref[1] + 1e-16
    rinv = lax.dot_general(1.0 / d, st_ref[...],
                           (((1,), (0,)), ((), ())),
                           preferred_element_type=_F32)
    cq = pcq_ref[...] / DISTANCE_SCALING
    wo = wo_ref[...]
    parts = []
    for i in range(4):
        r = (acc_ref[i, 0] + acc_ref[i, 1]) * rinv
        if i > 0:
            r = r - cq[:, i - 1][:, None]
        o = lax.dot_general(r, wo, (((1,), (0,)), ((), ())),
                            preferred_element_type=_F32)
        parts.append(o[:, None, :])
    out_ref[...] = jnp.concatenate(parts, axis=1)


def _finalize(den2, acc4, pcq, st, wo):
    bn = 1000
    return pl.pallas_call(
        _final_body,
        grid=(N // bn,),
        in_specs=[pl.BlockSpec((2, bn, 16), lambda i: (0, i, 0)),
                  pl.BlockSpec((4, 2, bn, 128), lambda i: (0, 0, i, 0)),
                  pl.BlockSpec((bn, 3), lambda i: (i, 0)),
                  pl.BlockSpec((16, 128), lambda i: (0, 0)),
                  pl.BlockSpec((128, 128), lambda i: (0, 0))],
        out_specs=pl.BlockSpec((bn, 4, 128), lambda i: (i, 0, 0)),
        out_shape=jax.ShapeDtypeStruct((N, 4, 128), _F32),
    )(den2, acc4, pcq, st, wo)


def kernel(x_k, x_q, edge_index, point_centers_k, point_centers_q, x_edge,
           Wq, Wk, Wv, We, Wo, point_weights):
    perm = jnp.array([h * POINT_DIM + p
                      for p in range(POINT_DIM) for h in range(HEADS)],
                     dtype=jnp.int32)
    wq_p = Wq[:, perm]
    wk_p = Wk[:, perm]
    wv_p = Wv[:, perm]
    wo_p = Wo[perm, :]
    pw = jax.nn.softplus(point_weights)
    c2 = (jnp.tile(pw, POINT_DIM) * POINT_SCALE).reshape(1, 128)
    sel = jnp.tile(jnp.eye(16, dtype=_F32), (POINT_DIM, 1))

    alo, ahi, blo, bhi, nk, *vs = _prep(
        x_q, x_k, point_centers_q, point_centers_k, wq_p, wk_p, wv_p, c2, sel)
    eb = _ebias(x_edge, We)

    src = edge_index[0]
    dst = edge_index[1]
    zd = jnp.zeros((NPS, 16), _F32)
    za = jnp.zeros((NPS, 128), _F32)
    den2, acc4, _ex = _sc_edge(
        src, dst, alo.reshape(N, 256), ahi.reshape(N, 256),
        blo.reshape(N, 256), bhi.reshape(N, 256), nk, eb, vs, zd, za)

    return _finalize(den2[:, :N], acc4[:, :, :N], point_centers_q, sel.T, wo_p)


# speedup vs baseline: 26.6815x; 1.0102x over previous
import jax
import jax.numpy as jnp
from jax import lax
from jax.experimental import pallas as pl
from jax.experimental.pallas import tpu as pltpu
from jax.experimental.pallas import tpu_sc as plsc

FIBER_DIM = 128
HEADS = 16
POINT_DIM = 8
N = 10000
E = 320000
DISTANCE_SCALING = 10.0
SCALAR_SCALE = (2 * POINT_DIM) ** (-0.5)
POINT_SCALE = (2 * POINT_DIM * (9.0 / 2.0)) ** (-0.5)

NC = 2
NS = 16
NW = NC * NS
EPW = E // NW
BE = 40
NB = EPW // BE
BB = 40
NBB = EPW // BB
NP_ = 10240
NPS = NP_ // NS

_F32 = jnp.float32


def _prep_body(xq_ref, xk_ref, pcq_ref, pck_ref, wq_ref, wk_ref, wv_ref,
               c2_ref, s_ref, alo_ref, ahi_ref, blo_ref, bhi_ref, nk_ref,
               *v_refs):
    bn = xq_ref.shape[0]
    wq = wq_ref[...]
    wk = wk_ref[...]
    wv = wv_ref[...]
    c2 = c2_ref[...]
    sel = s_ref[...]

    def mm(x, w):
        return lax.dot_general(x.reshape(bn * 4, 128), w,
                               (((1,), (0,)), ((), ())),
                               preferred_element_type=_F32).reshape(bn, 4, 128)

    q = mm(xq_ref[...], wq)
    k = mm(xk_ref[...], wk)
    v = mm(xk_ref[...], wv)
    cq = pcq_ref[...] / DISTANCE_SCALING
    ck = pck_ref[...] / DISTANCE_SCALING

    sq = q[:, 0, :]
    pq = q[:, 1:, :] + cq[:, :, None]
    sk = k[:, 0, :]
    pk = k[:, 1:, :] + ck[:, :, None]
    sv = v[:, 0, :]
    pv = v[:, 1:, :] + ck[:, :, None]

    ap = pq * c2[None, :, :]
    alo_ref[:, 0:1, :] = (sq * SCALAR_SCALE)[:, None, :]
    alo_ref[:, 1:2, :] = ap[:, 0:1, :]
    ahi_ref[...] = ap[:, 1:3, :]
    blo_ref[:, 0:1, :] = sk[:, None, :]
    blo_ref[:, 1:2, :] = pk[:, 0:1, :]
    bhi_ref[...] = pk[:, 1:3, :]

    uk = (pk * pk) * c2[None, :, :]
    nk_ref[...] = 0.5 * lax.dot_general(uk[:, 0, :] + uk[:, 1, :] + uk[:, 2, :],
                                        sel, (((1,), (0,)), ((), ())),
                                        preferred_element_type=_F32)
    vs = (sv, pv[:, 0, :], pv[:, 1, :], pv[:, 2, :])
    for i in range(4):
        v_refs[i][...] = vs[i]


def _prep(xq, xk, pcq, pck, wq, wk, wv, c2, sel):
    bn = 1000
    grid = N // bn
    full = lambda shp: pl.BlockSpec(shp, lambda i: tuple(0 for _ in shp))
    row3 = pl.BlockSpec((bn, 4, 128), lambda i: (i, 0, 0))
    return pl.pallas_call(
        _prep_body,
        grid=(grid,),
        in_specs=[row3, row3,
                  pl.BlockSpec((bn, 3), lambda i: (i, 0)),
                  pl.BlockSpec((bn, 3), lambda i: (i, 0)),
                  full((128, 128)), full((128, 128)), full((128, 128)),
                  full((1, 128)), full((128, 16))],
        out_specs=[pl.BlockSpec((bn, 2, 128), lambda i: (i, 0, 0))] * 4 +
                  [pl.BlockSpec((bn, 16), lambda i: (i, 0))] +
                  [pl.BlockSpec((bn, 128), lambda i: (i, 0))] * 4,
        out_shape=[jax.ShapeDtypeStruct((N, 2, 128), _F32)] * 4 +
                  [jax.ShapeDtypeStruct((N, 16), _F32)] +
                  [jax.ShapeDtypeStruct((N, 128), _F32)] * 4,
    )(xq, xk, pcq, pck, wq, wk, wv, c2, sel)


def _ebias_body(xe_ref, we_ref, out_ref):
    out_ref[...] = lax.dot_general(xe_ref[...], we_ref[...],
                                   (((1,), (0,)), ((), ())),
                                   preferred_element_type=_F32)


def _ebias(x_edge, we):
    be = 8000
    return pl.pallas_call(
        _ebias_body,
        grid=(E // be,),
        in_specs=[pl.BlockSpec((be, 16), lambda i: (i, 0)),
                  pl.BlockSpec((16, 16), lambda i: (0, 0))],
        out_specs=pl.BlockSpec((be, 16), lambda i: (i, 0)),
        out_shape=jax.ShapeDtypeStruct((E, 16), _F32),
    )(x_edge, we)


def _sc_body(src_h, dst_h, alo_h, ahi_h, blo_h, bhi_h, nk_h, eb_h,
             v0_h, v1_h, v2_h, v3_h, zd_h, za_h,
             den_out, acc_out, ex_out,
             ixs0, ixd0, nkr, ebr, exb,
             den_sp, acc_sp, sem0, sem1, sem2):
    c = lax.axis_index("c")
    s = lax.axis_index("s")
    wid = s * NC + c
    ebase = wid * EPW
    rbase = s * NPS

    def main(arows, brows, vr0, vr1, plog, jxs0, jxd0, jxs1, jxd1,
             fxq0, fxq1):
        pltpu.sync_copy(zd_h, den_sp.at[pl.ds(rbase, NPS)])
        plsc.subcore_barrier()

        def blk_a(i, carry):
            base = ebase + i * BE
            pltpu.sync_copy(src_h.at[pl.ds(base, BE)], ixs0)
            pltpu.sync_copy(dst_h.at[pl.ds(base, BE)], ixd0)
            cp_a = pltpu.async_copy(alo_h.at[ixd0], arows, sem0)
            cp_b = pltpu.async_copy(blo_h.at[ixs0], brows, sem1)
            cp_nk = pltpu.async_copy(nk_h.at[ixs0], nkr, sem2)
            pltpu.sync_copy(eb_h.at[pl.ds(base, BE)], ebr)
            cp_a.wait()
            cp_b.wait()

            @plsc.parallel_loop(0, BE)
            def edge_lo(j):
                ps = [arows[j, pl.ds(0, 16)] * brows[j, pl.ds(0, 16)],
                      arows[j, pl.ds(16, 16)] * brows[j, pl.ds(16, 16)],
                      arows[j, pl.ds(32, 16)] * brows[j, pl.ds(32, 16)],
                      arows[j, pl.ds(48, 16)] * brows[j, pl.ds(48, 16)]]
                for t in range(4, 16):
                    o = t * 16
                    ps[t % 4] = ps[t % 4] + arows[j, pl.ds(o, 16)] * brows[j, pl.ds(o, 16)]
                plog[j] = (ps[0] + ps[1]) + (ps[2] + ps[3])

            cp_a2 = pltpu.async_copy(ahi_h.at[ixd0], arows, sem0)
            cp_b2 = pltpu.async_copy(bhi_h.at[ixs0], brows, sem1)
            cp_nk.wait()
            cp_a2.wait()
            cp_b2.wait()

            @plsc.parallel_loop(0, BE)
            def edge_hi(j):
                ps = [plog[j] + ebr[j] - nkr[j],
                      arows[j, pl.ds(0, 16)] * brows[j, pl.ds(0, 16)],
                      arows[j, pl.ds(16, 16)] * brows[j, pl.ds(16, 16)],
                      arows[j, pl.ds(32, 16)] * brows[j, pl.ds(32, 16)]]
                for t in range(3, 16):
                    o = t * 16
                    ps[t % 4] = ps[t % 4] + arows[j, pl.ds(o, 16)] * brows[j, pl.ds(o, 16)]
                exb[j] = jnp.exp((ps[0] + ps[1]) + (ps[2] + ps[3]))

            pltpu.sync_copy(exb, ex_out.at[pl.ds(base, BE)])
            pltpu.sync_copy(exb, den_sp.at[ixd0], add=True)
            return carry

        lax.fori_loop(0, NB, blk_a, 0, unroll=False)
        plsc.subcore_barrier()
        pltpu.sync_copy(den_sp.at[pl.ds(rbase, NPS)],
                        den_out.at[c, pl.ds(rbase, NPS)])

        bufs = ((vr0, jxs0, jxd0, fxq0, sem0), (vr1, jxs1, jxd1, fxq1, sem1))
        for part, v_h in enumerate((v0_h, v1_h, v2_h, v3_h)):
            plsc.subcore_barrier()
            pltpu.sync_copy(za_h, acc_sp.at[pl.ds(rbase, NPS)])
            plsc.subcore_barrier()

            pltpu.sync_copy(src_h.at[pl.ds(ebase, BB)], jxs0)
            pltpu.sync_copy(dst_h.at[pl.ds(ebase, BB)], jxd0)
            pltpu.sync_copy(ex_out.at[pl.ds(ebase, BB)], fxq0)
            pltpu.async_copy(v_h.at[jxs0], vr0, sem0)

            def step(bufcur, bufnxt, i, prefetch):
                vr, ixs, ixd, exq, sem = bufcur
                vrn, ixsn, ixdn, exqn, semn = bufnxt
                if prefetch:
                    nxt = ebase + (i + 1) * BB
                    pltpu.sync_copy(src_h.at[pl.ds(nxt, BB)], ixsn)
                    pltpu.sync_copy(dst_h.at[pl.ds(nxt, BB)], ixdn)
                    pltpu.sync_copy(ex_out.at[pl.ds(nxt, BB)], exqn)
                    pltpu.async_copy(v_h.at[ixsn], vrn, semn)
                pltpu.make_async_copy(v_h.at[ixs], vr, sem).wait()

                @plsc.parallel_loop(0, BB)
                def edge(j):
                    al = exq[j]
                    for t in range(8):
                        o = t * 16
                        vr[j, pl.ds(o, 16)] = vr[j, pl.ds(o, 16)] * al

                pltpu.sync_copy(vr, acc_sp.at[ixd], add=True)

            def blk2(g, carry):
                for b in (0, 1):
                    step(bufs[b], bufs[1 - b], 2 * g + b, True)
                return carry

            npair = (NBB - 1) // 2
            lax.fori_loop(0, npair, blk2, 0, unroll=False)
            for r in range(NBB - 2 * npair):
                k = 2 * npair + r
                step(bufs[k % 2], bufs[(k + 1) % 2], k,
                     r < NBB - 2 * npair - 1)

            plsc.subcore_barrier()
            pltpu.sync_copy(acc_sp.at[pl.ds(rbase, NPS)],
                            acc_out.at[part, c, pl.ds(rbase, NPS)])
        plsc.subcore_barrier()

    pl.run_scoped(main,
                  pltpu.VMEM((BE, 256), _F32), pltpu.VMEM((BE, 256), _F32),
                  pltpu.VMEM((BB, 128), _F32), pltpu.VMEM((BB, 128), _F32),
                  pltpu.VMEM((BE, 16), _F32),
                  pltpu.VMEM((BB,), jnp.int32), pltpu.VMEM((BB,), jnp.int32),
                  pltpu.VMEM((BB,), jnp.int32), pltpu.VMEM((BB,), jnp.int32),
                  pltpu.VMEM((BB, 16), _F32), pltpu.VMEM((BB, 16), _F32))


def _sc_edge(src, dst, alo, ahi, blo, bhi, nk, eb, vs, zd, za):
    mesh = plsc.VectorSubcoreMesh(core_axis_name="c", subcore_axis_name="s")
    f = pl.kernel(
        _sc_body,
        out_type=[jax.ShapeDtypeStruct((NC, NP_, 16), _F32),
                  jax.ShapeDtypeStruct((4, NC, NP_, 128), _F32),
                  jax.ShapeDtypeStruct((E, 16), _F32)],
        mesh=mesh,
        compiler_params=pltpu.CompilerParams(use_tc_tiling_on_sc=False),
        scratch_types=[
            pltpu.VMEM((BE,), jnp.int32), pltpu.VMEM((BE,), jnp.int32),
            pltpu.VMEM((BE, 16), _F32), pltpu.VMEM((BE, 16), _F32),
            pltpu.VMEM((BE, 16), _F32),
            pltpu.VMEM_SHARED((NP_, 16), _F32),
            pltpu.VMEM_SHARED((NP_, 128), _F32),
            pltpu.SemaphoreType.DMA,
            pltpu.SemaphoreType.DMA,
            pltpu.SemaphoreType.DMA,
        ],
    )
    return f(src, dst, alo, ahi, blo, bhi, nk, eb, *vs, zd, za)


def _final_body(den_ref, acc_ref, pcq_ref, st_ref, wo_ref, out_ref):
    bn = pcq_ref.shape[0]
    d = den_ref[0] + den_ref[1] + 1e-16
    rinv = lax.dot_general(1.0 / d, st_ref[...],
                           (((1,), (0,)), ((), ())),
                           preferred_element_type=_F32)
    cq = pcq_ref[...] / DISTANCE_SCALING
    wo = wo_ref[...]
    parts = []
    for i in range(4):
        r = (acc_ref[i, 0] + acc_ref[i, 1]) * rinv
        if i > 0:
            r = r - cq[:, i - 1][:, None]
        o = lax.dot_general(r, wo, (((1,), (0,)), ((), ())),
                            preferred_element_type=_F32)
        parts.append(o[:, None, :])
    out_ref[...] = jnp.concatenate(parts, axis=1)


def _finalize(den2, acc4, pcq, st, wo):
    bn = 1000
    return pl.pallas_call(
        _final_body,
        grid=(N // bn,),
        in_specs=[pl.BlockSpec((2, bn, 16), lambda i: (0, i, 0)),
                  pl.BlockSpec((4, 2, bn, 128), lambda i: (0, 0, i, 0)),
                  pl.BlockSpec((bn, 3), lambda i: (i, 0)),
                  pl.BlockSpec((16, 128), lambda i: (0, 0)),
                  pl.BlockSpec((128, 128), lambda i: (0, 0))],
        out_specs=pl.BlockSpec((bn, 4, 128), lambda i: (i, 0, 0)),
        out_shape=jax.ShapeDtypeStruct((N, 4, 128), _F32),
    )(den2, acc4, pcq, st, wo)


def kernel(x_k, x_q, edge_index, point_centers_k, point_centers_q, x_edge,
           Wq, Wk, Wv, We, Wo, point_weights):
    perm = jnp.array([h * POINT_DIM + p
                      for p in range(POINT_DIM) for h in range(HEADS)],
                     dtype=jnp.int32)
    wq_p = Wq[:, perm]
    wk_p = Wk[:, perm]
    wv_p = Wv[:, perm]
    wo_p = Wo[perm, :]
    pw = jax.nn.softplus(point_weights)
    c2 = (jnp.tile(pw, POINT_DIM) * POINT_SCALE).reshape(1, 128)
    sel = jnp.tile(jnp.eye(16, dtype=_F32), (POINT_DIM, 1))

    alo, ahi, blo, bhi, nk, *vs = _prep(
        x_q, x_k, point_centers_q, point_centers_k, wq_p, wk_p, wv_p, c2, sel)
    eb = _ebias(x_edge, We)

    src = edge_index[0]
    dst = edge_index[1]
    zd = jnp.zeros((NPS, 16), _F32)
    za = jnp.zeros((NPS, 128), _F32)
    den2, acc4, _ex = _sc_edge(
        src, dst, alo.reshape(N, 256), ahi.reshape(N, 256),
        blo.reshape(N, 256), bhi.reshape(N, 256), nk, eb, vs, zd, za)

    return _finalize(den2, acc4, point_centers_q, sel.T, wo_p)
